# Initial kernel scaffold; baseline (speedup 1.0000x reference)
import jax
import jax.numpy as jnp
from jax.experimental import pallas as pl


def kernel(x, edge_index, edge_attr, u, batch, params):
    raise NotImplementedError("write your pallas kernel here")


# speedup vs baseline: 1.3361x; 1.3361x over previous
import functools

import jax
import jax.numpy as jnp
from jax import lax
from jax.experimental import pallas as pl
from jax.experimental.pallas import tpu as pltpu
from jax.experimental.pallas import tpu_sc as plsc

_BN_EPS = 1e-5
_NC = 2
_NS = 16
_C = 128


def _leaky(v):
    return jnp.where(v >= 0, v, 0.1 * v)


def _linear_body(nin, act, has_head, has_stats, *refs):
    ins = refs[:nin]
    ws = refs[nin:2 * nin]
    b_ref = refs[2 * nin]
    k = 2 * nin + 1
    if has_head:
        hw_ref, hb_ref = refs[k], refs[k + 1]
        k += 2
    out_ref = refs[k]
    k += 1
    if has_head:
        hout_ref = refs[k]
        k += 1
    if has_stats:
        st_ref = refs[k]
    acc = jnp.dot(ins[0][...], ws[0][...], preferred_element_type=jnp.float32)
    for j in range(1, nin):
        acc = acc + jnp.dot(ins[j][...], ws[j][...],
                            preferred_element_type=jnp.float32)
    acc = acc + b_ref[...]
    if act:
        acc = _leaky(acc)
    out_ref[...] = acc
    if has_head:
        h = jnp.dot(acc, hw_ref[...], preferred_element_type=jnp.float32)
        hout_ref[...] = jax.nn.sigmoid(h + hb_ref[...])
    if has_stats:
        @pl.when(pl.program_id(0) == 0)
        def _init():
            st_ref[...] = jnp.zeros_like(st_ref)
        st_ref[0:1, :] += jnp.sum(acc, axis=0, keepdims=True)
        st_ref[1:2, :] += jnp.sum(acc * acc, axis=0, keepdims=True)


def _pick_block(m):
    for r in (4000, 2000, 1000):
        if m % r == 0:
            return r
    raise ValueError(m)


def _linear(ins, ws, b, act, head=None, stats=False):
    m = ins[0].shape[0]
    dout = ws[0].shape[1]
    r = _pick_block(m)
    in_specs = [pl.BlockSpec((r, a.shape[1]), lambda i: (i, 0)) for a in ins]
    in_specs += [pl.BlockSpec(w.shape, lambda i: (0, 0)) for w in ws]
    in_specs += [pl.BlockSpec((1, dout), lambda i: (0, 0))]
    args = list(ins) + list(ws) + [b.reshape(1, dout)]
    out_shape = [jax.ShapeDtypeStruct((m, dout), jnp.float32)]
    out_specs = [pl.BlockSpec((r, dout), lambda i: (i, 0))]
    if head is not None:
        hw, hb = head
        in_specs += [pl.BlockSpec(hw.shape, lambda i: (0, 0)),
                     pl.BlockSpec((1, 1), lambda i: (0, 0))]
        args += [hw, hb.reshape(1, 1)]
        out_shape.append(jax.ShapeDtypeStruct((m, 1), jnp.float32))
        out_specs.append(pl.BlockSpec((r, 1), lambda i: (i, 0)))
    if stats:
        out_shape.append(jax.ShapeDtypeStruct((8, dout), jnp.float32))
        out_specs.append(pl.BlockSpec((8, dout), lambda i: (0, 0)))
    fn = pl.pallas_call(
        functools.partial(_linear_body, len(ins), act, head is not None, stats),
        grid=(m // r,),
        in_specs=in_specs,
        out_specs=out_specs,
        out_shape=out_shape,
        compiler_params=pltpu.CompilerParams(
            dimension_semantics=("arbitrary",)),
    )
    return fn(*args)


def _stats_body(nin, *refs):
    ins = refs[:nin]
    sts = refs[nin:]

    @pl.when(pl.program_id(0) == 0)
    def _init():
        for s in sts:
            s[...] = jnp.zeros_like(s)

    for rr, s in zip(ins, sts):
        v = rr[...]
        s[0:1, :] += jnp.sum(v, axis=0, keepdims=True)
        s[1:2, :] += jnp.sum(v * v, axis=0, keepdims=True)


def _col_stats(ins):
    m = ins[0].shape[0]
    r = _pick_block(m)
    in_specs = [pl.BlockSpec((r, a.shape[1]), lambda i: (i, 0)) for a in ins]
    out_shape = [jax.ShapeDtypeStruct((8, a.shape[1]), jnp.float32) for a in ins]
    out_specs = [pl.BlockSpec((8, a.shape[1]), lambda i: (0, 0)) for a in ins]
    fn = pl.pallas_call(
        functools.partial(_stats_body, len(ins)),
        grid=(m // r,),
        in_specs=in_specs,
        out_specs=out_specs,
        out_shape=out_shape,
        compiler_params=pltpu.CompilerParams(
            dimension_semantics=("arbitrary",)),
    )
    return fn(*ins)


def _fold_bn(w, b, g, bb, stats_list, n):
    m = jnp.concatenate([s[0] / n for s in stats_list])
    msq = jnp.concatenate([s[1] / n for s in stats_list])
    v = msq - m * m
    scale = g / jnp.sqrt(v + _BN_EPS)
    weff = w * scale[:, None]
    beff = b + (bb - m * scale) @ w
    return weff, beff


def _split_rows(w, dims):
    out, o = [], 0
    for d in dims:
        out.append(w[o:o + d])
        o += d
    return out


def _sc_gather(x, row, col):
    n, f = x.shape
    e = row.shape[0]
    nchunk = e // _C
    nw = _NC * _NS
    mesh = plsc.VectorSubcoreMesh(core_axis_name="c", subcore_axis_name="s")

    @functools.partial(
        pl.kernel,
        out_type=(jax.ShapeDtypeStruct((e, f), jnp.float32),
                  jax.ShapeDtypeStruct((e, f), jnp.float32)),
        mesh=mesh,
        scratch_types=[pltpu.VMEM((_C,), jnp.int32),
                       pltpu.VMEM((_C,), jnp.int32),
                       pltpu.VMEM((_C, f), jnp.float32),
                       pltpu.VMEM((_C, f), jnp.float32),
                       pltpu.SemaphoreType.DMA,
                       pltpu.SemaphoreType.DMA],
        compiler_params=pltpu.CompilerParams(use_tc_tiling_on_sc=False),
    )
    def k(x_hbm, row_hbm, col_hbm, xr_hbm, xc_hbm, idr, idc, bufr, bufc, s1, s2):
        cid = lax.axis_index("c")
        sid = lax.axis_index("s")
        wid = sid * _NC + cid
        nloc = (nchunk - wid + nw - 1) // nw

        def body(t, carry):
            base = (wid + t * nw) * _C
            pltpu.sync_copy(row_hbm.at[pl.ds(base, _C)], idr)
            pltpu.sync_copy(col_hbm.at[pl.ds(base, _C)], idc)
            cp1 = pltpu.async_copy(x_hbm.at[idr], bufr, s1)
            cp2 = pltpu.async_copy(x_hbm.at[idc], bufc, s2)
            cp1.wait()
            cp2.wait()
            pltpu.sync_copy(bufr, xr_hbm.at[pl.ds(base, _C)])
            pltpu.sync_copy(bufc, xc_hbm.at[pl.ds(base, _C)])
            return carry

        lax.fori_loop(0, nloc, body, 0)

    return k(x, row, col)


def _sc_scatter_add(hn, col, n):
    e, d = hn.shape
    h = d // _NC
    nchunk = e // _C
    rps = n // _NS
    rb = 625
    zeros = jnp.zeros((rb, h), jnp.float32)
    mesh = plsc.VectorSubcoreMesh(core_axis_name="c", subcore_axis_name="s")

    @functools.partial(
        pl.kernel,
        out_type=jax.ShapeDtypeStruct((n, d), jnp.float32),
        mesh=mesh,
        scratch_types=[pltpu.VMEM_SHARED((n, h), jnp.float32),
                       pltpu.VMEM((_C,), jnp.int32),
                       pltpu.VMEM((_C, h), jnp.float32),
                       pltpu.VMEM((rb, h), jnp.float32)],
        compiler_params=pltpu.CompilerParams(use_tc_tiling_on_sc=False),
    )
    def k(hn_hbm, col_hbm, z_hbm, agg_hbm, shared, idx, buf, rbuf):
        cid = lax.axis_index("c")
        sid = lax.axis_index("s")
        pltpu.sync_copy(z_hbm, rbuf)

        def zb(t, carry):
            pltpu.sync_copy(rbuf, shared.at[pl.ds(sid * rps + t * rb, rb)])
            return carry

        lax.fori_loop(0, rps // rb, zb, 0)
        plsc.subcore_barrier()

        nloc = (nchunk - sid + _NS - 1) // _NS

        def body(t, carry):
            base = (sid + t * _NS) * _C
            pltpu.sync_copy(col_hbm.at[pl.ds(base, _C)], idx)
            pltpu.sync_copy(hn_hbm.at[pl.ds(base, _C), pl.ds(cid * h, h)], buf)
            pltpu.sync_copy(buf, shared.at[idx], add=True)
            return carry

        lax.fori_loop(0, nloc, body, 0)
        plsc.subcore_barrier()

        def wb(t, carry):
            r0 = sid * rps + t * rb
            pltpu.sync_copy(shared.at[pl.ds(r0, rb)], rbuf)
            pltpu.sync_copy(rbuf, agg_hbm.at[pl.ds(r0, rb), pl.ds(cid * h, h)])
            return carry

        lax.fori_loop(0, rps // rb, wb, 0)

    return k(hn, col, zeros)


def kernel(x, edge_index, edge_attr, u, batch, params):
    del u, batch
    row, col = edge_index[0], edge_index[1]
    e = row.shape[0]
    n = x.shape[0]
    n_e = jnp.float32(e)
    n_n = jnp.float32(n)
    ea = edge_attr
    (ea_st,) = _col_stats([ea])
    (x_st,) = _col_stats([x])
    y_pred = edge_pred = None
    for i in range(3):
        p = params['ml%d' % i]
        xr, xc = _sc_gather(x, row, col)
        xr_st, xc_st = _col_stats([xr, xc])
        pe = p['edge']
        w1, b1 = _fold_bn(pe['W1'], pe['b1'], pe['g1'], pe['bb1'],
                          [xr_st, xc_st, ea_st], n_e)
        w1a, w1b, w1c = _split_rows(w1, [16, 16, 19])
        h1, h1_st = _linear([xr, xc, ea], [w1a, w1b, w1c], b1, act=True,
                            stats=True)
        w2, b2 = _fold_bn(pe['W2'], pe['b2'], pe['g2'], pe['bb2'], [h1_st], n_e)
        h2, h2_st = _linear([h1], [w2], b2, act=True, stats=True)
        w3, b3 = _fold_bn(pe['W3'], pe['b3'], pe['g3'], pe['bb3'], [h2_st], n_e)
        if i == 2:
            ea, edge_pred, ea_st = _linear([h2], [w3], b3, act=False,
                                           stats=True,
                                           head=(params['eW'], params['eb']))
        else:
            ea, ea_st = _linear([h2], [w3], b3, act=False, stats=True)
        pn1 = p['n1']
        v1, c1 = _fold_bn(pn1['W1'], pn1['b1'], pn1['g1'], pn1['bb1'],
                          [xr_st, ea_st], n_e)
        v1a, v1b = _split_rows(v1, [16, 19])
        g1, g1_st = _linear([xr, ea], [v1a, v1b], c1, act=True, stats=True)
        v2, c2 = _fold_bn(pn1['W2'], pn1['b2'], pn1['g2'], pn1['bb2'],
                          [g1_st], n_e)
        g2, g2_st = _linear([g1], [v2], c2, act=True, stats=True)
        v3, c3 = _fold_bn(pn1['W3'], pn1['b3'], pn1['g3'], pn1['bb3'],
                          [g2_st], n_e)
        (hn,) = _linear([g2], [v3], c3, act=False, stats=False)
        agg = _sc_scatter_add(hn, col, n)
        (agg_st,) = _col_stats([agg])
        pn2 = p['n2']
        u1w, d1 = _fold_bn(pn2['W1'], pn2['b1'], pn2['g1'], pn2['bb1'],
                           [x_st, agg_st], n_n)
        u1a, u1b = _split_rows(u1w, [16, 64])
        t1, t1_st = _linear([x, agg], [u1a, u1b], d1, act=True, stats=True)
        u2w, d2 = _fold_bn(pn2['W2'], pn2['b2'], pn2['g2'], pn2['bb2'],
                           [t1_st], n_n)
        t2, t2_st = _linear([t1], [u2w], d2, act=True, stats=True)
        u3w, d3 = _fold_bn(pn2['W3'], pn2['b3'], pn2['g3'], pn2['bb3'],
                           [t2_st], n_n)
        if i == 2:
            x, y_pred, x_st = _linear([t2], [u3w], d3, act=False, stats=True,
                                      head=(params['xW'], params['xb']))
        else:
            x, x_st = _linear([t2], [u3w], d3, act=False, stats=True)
    return (y_pred, edge_pred)


# speedup vs baseline: 1.6125x; 1.2068x over previous
import functools

import jax
import jax.numpy as jnp
from jax import lax
from jax.experimental import pallas as pl
from jax.experimental.pallas import tpu as pltpu
from jax.experimental.pallas import tpu_sc as plsc

_BN_EPS = 1e-5
_NC = 2
_NS = 16
_C = 128


def _leaky(v):
    return jnp.where(v >= 0, v, 0.1 * v)


def _linear_body(nin, act, has_head, has_stats, *refs):
    ins = refs[:nin]
    ws = refs[nin:2 * nin]
    b_ref = refs[2 * nin]
    k = 2 * nin + 1
    if has_head:
        hw_ref, hb_ref = refs[k], refs[k + 1]
        k += 2
    out_ref = refs[k]
    k += 1
    if has_head:
        hout_ref = refs[k]
        k += 1
    if has_stats:
        st_ref = refs[k]
    acc = jnp.dot(ins[0][...], ws[0][...], preferred_element_type=jnp.float32)
    for j in range(1, nin):
        acc = acc + jnp.dot(ins[j][...], ws[j][...],
                            preferred_element_type=jnp.float32)
    acc = acc + b_ref[...]
    if act:
        acc = _leaky(acc)
    out_ref[...] = acc
    if has_head:
        h = jnp.dot(acc, hw_ref[...], preferred_element_type=jnp.float32)
        hout_ref[...] = jax.nn.sigmoid(h + hb_ref[...])
    if has_stats:
        @pl.when(pl.program_id(0) == 0)
        def _init():
            st_ref[...] = jnp.zeros_like(st_ref)
        st_ref[0:1, :] += jnp.sum(acc, axis=0, keepdims=True)
        st_ref[1:2, :] += jnp.sum(acc * acc, axis=0, keepdims=True)


def _pick_block(m):
    for r in (4000, 2000, 1000):
        if m % r == 0:
            return r
    raise ValueError(m)


def _linear(ins, ws, b, act, head=None, stats=False):
    m = ins[0].shape[0]
    dout = ws[0].shape[1]
    r = _pick_block(m)
    in_specs = [pl.BlockSpec((r, a.shape[1]), lambda i: (i, 0)) for a in ins]
    in_specs += [pl.BlockSpec(w.shape, lambda i: (0, 0)) for w in ws]
    in_specs += [pl.BlockSpec((1, dout), lambda i: (0, 0))]
    args = list(ins) + list(ws) + [b.reshape(1, dout)]
    out_shape = [jax.ShapeDtypeStruct((m, dout), jnp.float32)]
    out_specs = [pl.BlockSpec((r, dout), lambda i: (i, 0))]
    if head is not None:
        hw, hb = head
        in_specs += [pl.BlockSpec(hw.shape, lambda i: (0, 0)),
                     pl.BlockSpec((1, 1), lambda i: (0, 0))]
        args += [hw, hb.reshape(1, 1)]
        out_shape.append(jax.ShapeDtypeStruct((m, 1), jnp.float32))
        out_specs.append(pl.BlockSpec((r, 1), lambda i: (i, 0)))
    if stats:
        out_shape.append(jax.ShapeDtypeStruct((8, dout), jnp.float32))
        out_specs.append(pl.BlockSpec((8, dout), lambda i: (0, 0)))
    fn = pl.pallas_call(
        functools.partial(_linear_body, len(ins), act, head is not None, stats),
        grid=(m // r,),
        in_specs=in_specs,
        out_specs=out_specs,
        out_shape=out_shape,
        compiler_params=pltpu.CompilerParams(
            dimension_semantics=("arbitrary",)),
    )
    return fn(*args)


def _stats_body(nin, *refs):
    ins = refs[:nin]
    sts = refs[nin:]

    @pl.when(pl.program_id(0) == 0)
    def _init():
        for s in sts:
            s[...] = jnp.zeros_like(s)

    for rr, s in zip(ins, sts):
        v = rr[...]
        s[0:1, :] += jnp.sum(v, axis=0, keepdims=True)
        s[1:2, :] += jnp.sum(v * v, axis=0, keepdims=True)


def _col_stats(ins):
    m = ins[0].shape[0]
    r = _pick_block(m)
    in_specs = [pl.BlockSpec((r, a.shape[1]), lambda i: (i, 0)) for a in ins]
    out_shape = [jax.ShapeDtypeStruct((8, a.shape[1]), jnp.float32) for a in ins]
    out_specs = [pl.BlockSpec((8, a.shape[1]), lambda i: (0, 0)) for a in ins]
    fn = pl.pallas_call(
        functools.partial(_stats_body, len(ins)),
        grid=(m // r,),
        in_specs=in_specs,
        out_specs=out_specs,
        out_shape=out_shape,
        compiler_params=pltpu.CompilerParams(
            dimension_semantics=("arbitrary",)),
    )
    return fn(*ins)


def _fold_bn(w, b, g, bb, stats_list, n):
    m = jnp.concatenate([s[0] / n for s in stats_list])
    msq = jnp.concatenate([s[1] / n for s in stats_list])
    v = msq - m * m
    scale = g / jnp.sqrt(v + _BN_EPS)
    weff = w * scale[:, None]
    beff = b + (bb - m * scale) @ w
    return weff, beff


def _split_rows(w, dims):
    out, o = [], 0
    for d in dims:
        out.append(w[o:o + d])
        o += d
    return out


_K = 5


def _sc_gather(x, row2d, col2d):
    n, f = x.shape
    nchunk = row2d.shape[0]
    nsup = nchunk // _K
    ss = _K * _C
    nw = _NC * _NS
    e = nchunk * _C
    mesh = plsc.VectorSubcoreMesh(core_axis_name="c", subcore_axis_name="s")

    @functools.partial(
        pl.kernel,
        out_type=(jax.ShapeDtypeStruct((e, f), jnp.float32),
                  jax.ShapeDtypeStruct((e, f), jnp.float32)),
        mesh=mesh,
        scratch_types=[pltpu.VMEM((_K, _C), jnp.int32),
                       pltpu.VMEM((_K, _C), jnp.int32),
                       pltpu.VMEM((ss, f), jnp.float32),
                       pltpu.VMEM((ss, f), jnp.float32),
                       pltpu.SemaphoreType.DMA,
                       pltpu.SemaphoreType.DMA],
        compiler_params=pltpu.CompilerParams(use_tc_tiling_on_sc=False),
    )
    def k(x_hbm, row_hbm, col_hbm, xr_hbm, xc_hbm, idr, idc, bufr, bufc, s1, s2):
        cid = lax.axis_index("c")
        sid = lax.axis_index("s")
        wid = sid * _NC + cid
        nloc = (nsup - wid + nw - 1) // nw

        def body(t, carry):
            sc = wid + t * nw
            base = sc * ss
            pltpu.sync_copy(row_hbm.at[pl.ds(sc * _K, _K)], idr)
            pltpu.sync_copy(col_hbm.at[pl.ds(sc * _K, _K)], idc)
            cps = []
            for j in range(_K):
                cps.append(pltpu.async_copy(
                    x_hbm.at[idr.at[j]], bufr.at[pl.ds(j * _C, _C)], s1))
                cps.append(pltpu.async_copy(
                    x_hbm.at[idc.at[j]], bufc.at[pl.ds(j * _C, _C)], s2))
            for cp in cps:
                cp.wait()
            pltpu.sync_copy(bufr, xr_hbm.at[pl.ds(base, ss)])
            pltpu.sync_copy(bufc, xc_hbm.at[pl.ds(base, ss)])
            return carry

        lax.fori_loop(0, nloc, body, 0)

    return k(x, row2d, col2d)


def _sc_scatter_add(hn, col2d, n):
    e, d = hn.shape
    h = d // _NC
    nchunk = e // _C
    nsup = nchunk // _K
    ss = _K * _C
    rps = n // _NS
    rb = 125
    zeros = jnp.zeros((rb, h), jnp.float32)
    mesh = plsc.VectorSubcoreMesh(core_axis_name="c", subcore_axis_name="s")

    @functools.partial(
        pl.kernel,
        out_type=jax.ShapeDtypeStruct((n, d), jnp.float32),
        mesh=mesh,
        scratch_types=[pltpu.VMEM_SHARED((n, h), jnp.float32),
                       pltpu.VMEM((_K, _C), jnp.int32),
                       pltpu.VMEM((ss, h), jnp.float32),
                       pltpu.VMEM((rb, h), jnp.float32),
                       pltpu.SemaphoreType.DMA,
                       pltpu.SemaphoreType.DMA],
        compiler_params=pltpu.CompilerParams(use_tc_tiling_on_sc=False),
    )
    def k(hn_hbm, col_hbm, z_hbm, agg_hbm, shared, idx, buf, rbuf, s1, s2):
        cid = lax.axis_index("c")
        sid = lax.axis_index("s")
        pltpu.sync_copy(z_hbm, rbuf)

        def zb(t, carry):
            pltpu.sync_copy(rbuf, shared.at[pl.ds(sid * rps + t * rb, rb)])
            return carry

        lax.fori_loop(0, rps // rb, zb, 0)
        plsc.subcore_barrier()

        nloc = (nsup - sid + _NS - 1) // _NS

        def body(t, carry):
            sc = sid + t * _NS
            base = sc * ss
            pltpu.sync_copy(col_hbm.at[pl.ds(sc * _K, _K)], idx)
            pltpu.async_copy(
                hn_hbm.at[pl.ds(base, ss), pl.ds(cid * h, h)], buf, s1).wait()
            for j in range(_K):
                pltpu.sync_copy(buf.at[pl.ds(j * _C, _C)], shared.at[idx.at[j]],
                                add=True)
            return carry

        lax.fori_loop(0, nloc, body, 0)
        plsc.subcore_barrier()

        def wb(t, carry):
            r0 = sid * rps + t * rb
            pltpu.sync_copy(shared.at[pl.ds(r0, rb)], rbuf)
            pltpu.sync_copy(rbuf, agg_hbm.at[pl.ds(r0, rb), pl.ds(cid * h, h)])
            return carry

        lax.fori_loop(0, rps // rb, wb, 0)

    return k(hn, col2d, zeros)


def _sc_degrees(row2d, col2d, n):
    nchunk = row2d.shape[0]
    nsup = nchunk // _K
    rps = n // _NS
    w = 16
    rb = 125
    zeros = jnp.zeros((rb, w), jnp.float32)
    ones = jnp.ones((_C, w), jnp.float32)
    mesh = plsc.VectorSubcoreMesh(core_axis_name="c", subcore_axis_name="s")

    @functools.partial(
        pl.kernel,
        out_type=jax.ShapeDtypeStruct((n, 2 * w), jnp.float32),
        mesh=mesh,
        scratch_types=[pltpu.VMEM_SHARED((n, w), jnp.float32),
                       pltpu.VMEM((_K, _C), jnp.int32),
                       pltpu.VMEM((_C, w), jnp.float32),
                       pltpu.VMEM((rb, w), jnp.float32)],
        compiler_params=pltpu.CompilerParams(use_tc_tiling_on_sc=False),
    )
    def k(row_hbm, col_hbm, z_hbm, one_hbm, deg_hbm, shared, idx, ones_v, rbuf):
        cid = lax.axis_index("c")
        sid = lax.axis_index("s")
        pltpu.sync_copy(z_hbm, rbuf)
        pltpu.sync_copy(one_hbm, ones_v)

        def zb(t, carry):
            pltpu.sync_copy(rbuf, shared.at[pl.ds(sid * rps + t * rb, rb)])
            return carry

        lax.fori_loop(0, rps // rb, zb, 0)
        plsc.subcore_barrier()
        nloc = (nsup - sid + _NS - 1) // _NS

        def body(t, carry):
            sc = sid + t * _NS

            @pl.when(cid == 0)
            def _r():
                pltpu.sync_copy(row_hbm.at[pl.ds(sc * _K, _K)], idx)

            @pl.when(cid == 1)
            def _c():
                pltpu.sync_copy(col_hbm.at[pl.ds(sc * _K, _K)], idx)

            for j in range(_K):
                pltpu.sync_copy(ones_v, shared.at[idx.at[j]], add=True)
            return carry

        lax.fori_loop(0, nloc, body, 0)
        plsc.subcore_barrier()

        def wb(t, carry):
            r0 = sid * rps + t * rb
            pltpu.sync_copy(shared.at[pl.ds(r0, rb)], rbuf)
            pltpu.sync_copy(rbuf, deg_hbm.at[pl.ds(r0, rb),
                                             pl.ds(cid * w, w)])
            return carry

        lax.fori_loop(0, rps // rb, wb, 0)

    return k(row2d, col2d, zeros, ones)


def _deg_stats_body(deg_ref, x_ref, out_ref):
    @pl.when(pl.program_id(0) == 0)
    def _init():
        out_ref[...] = jnp.zeros_like(out_ref)

    xv = x_ref[...]
    xcat = jnp.concatenate([xv, xv * xv], axis=1)
    out_ref[...] += lax.dot_general(
        deg_ref[...], xcat, (((0,), (0,)), ((), ())),
        preferred_element_type=jnp.float32)


def _deg_stats(degcat, x):
    m, f = x.shape
    r = _pick_block(m)
    fn = pl.pallas_call(
        _deg_stats_body,
        grid=(m // r,),
        in_specs=[pl.BlockSpec((r, 8), lambda i: (i, 0)),
                  pl.BlockSpec((r, f), lambda i: (i, 0))],
        out_specs=pl.BlockSpec((8, 2 * f), lambda i: (0, 0)),
        out_shape=jax.ShapeDtypeStruct((8, 2 * f), jnp.float32),
        compiler_params=pltpu.CompilerParams(
            dimension_semantics=("arbitrary",)),
    )
    out = fn(degcat, x)
    xr_st = jnp.stack([out[0, :f], out[0, f:]])
    xc_st = jnp.stack([out[1, :f], out[1, f:]])
    return xr_st, xc_st


def kernel(x, edge_index, edge_attr, u, batch, params):
    del u, batch
    row, col = edge_index[0], edge_index[1]
    e = row.shape[0]
    n = x.shape[0]
    n_e = jnp.float32(e)
    n_n = jnp.float32(n)
    row2d = row.reshape(e // _C, _C)
    col2d = col.reshape(e // _C, _C)
    ea = edge_attr
    (ea_st,) = _col_stats([ea])
    (x_st,) = _col_stats([x])
    degraw = _sc_degrees(row2d, col2d, n)
    degcat = jnp.concatenate([degraw[:, 0:1], degraw[:, 16:17],
                              jnp.zeros((n, 6), jnp.float32)], axis=1)
    y_pred = edge_pred = None
    for i in range(3):
        p = params['ml%d' % i]
        xr, xc = _sc_gather(x, row2d, col2d)
        xr_st, xc_st = _deg_stats(degcat, x)
        pe = p['edge']
        w1, b1 = _fold_bn(pe['W1'], pe['b1'], pe['g1'], pe['bb1'],
                          [xr_st, xc_st, ea_st], n_e)
        w1a, w1b, w1c = _split_rows(w1, [16, 16, 19])
        h1, h1_st = _linear([xr, xc, ea], [w1a, w1b, w1c], b1, act=True,
                            stats=True)
        w2, b2 = _fold_bn(pe['W2'], pe['b2'], pe['g2'], pe['bb2'], [h1_st], n_e)
        h2, h2_st = _linear([h1], [w2], b2, act=True, stats=True)
        w3, b3 = _fold_bn(pe['W3'], pe['b3'], pe['g3'], pe['bb3'], [h2_st], n_e)
        if i == 2:
            ea, edge_pred, ea_st = _linear([h2], [w3], b3, act=False,
                                           stats=True,
                                           head=(params['eW'], params['eb']))
        else:
            ea, ea_st = _linear([h2], [w3], b3, act=False, stats=True)
        pn1 = p['n1']
        v1, c1 = _fold_bn(pn1['W1'], pn1['b1'], pn1['g1'], pn1['bb1'],
                          [xr_st, ea_st], n_e)
        v1a, v1b = _split_rows(v1, [16, 19])
        g1, g1_st = _linear([xr, ea], [v1a, v1b], c1, act=True, stats=True)
        v2, c2 = _fold_bn(pn1['W2'], pn1['b2'], pn1['g2'], pn1['bb2'],
                          [g1_st], n_e)
        g2, g2_st = _linear([g1], [v2], c2, act=True, stats=True)
        v3, c3 = _fold_bn(pn1['W3'], pn1['b3'], pn1['g3'], pn1['bb3'],
                          [g2_st], n_e)
        (hn,) = _linear([g2], [v3], c3, act=False, stats=False)
        agg = _sc_scatter_add(hn, col2d, n)
        (agg_st,) = _col_stats([agg])
        pn2 = p['n2']
        u1w, d1 = _fold_bn(pn2['W1'], pn2['b1'], pn2['g1'], pn2['bb1'],
                           [x_st, agg_st], n_n)
        u1a, u1b = _split_rows(u1w, [16, 64])
        t1, t1_st = _linear([x, agg], [u1a, u1b], d1, act=True, stats=True)
        u2w, d2 = _fold_bn(pn2['W2'], pn2['b2'], pn2['g2'], pn2['bb2'],
                           [t1_st], n_n)
        t2, t2_st = _linear([t1], [u2w], d2, act=True, stats=True)
        u3w, d3 = _fold_bn(pn2['W3'], pn2['b3'], pn2['g3'], pn2['bb3'],
                           [t2_st], n_n)
        if i == 2:
            x, y_pred, x_st = _linear([t2], [u3w], d3, act=False, stats=True,
                                      head=(params['xW'], params['xb']))
        else:
            x, x_st = _linear([t2], [u3w], d3, act=False, stats=True)
    return (y_pred, edge_pred)
